# baseline (speedup 1.0000x reference)
import functools

import jax
import jax.numpy as jnp
from jax import lax
from jax.experimental import pallas as pl
from jax.experimental.pallas import tpu as pltpu
from jax.experimental.pallas import tpu_sc as plsc

NC = 2
NS = 16
NW = NC * NS

CHUNK = 125
IBLK = 8
DEG_W = 128


def _sc_aggregate(x, idx_r, zeros, *, n_pad, n_iter):
  in_ch = x.shape[1]
  rows_per_sub = n_pad // NS
  n_outer = n_iter // IBLK

  mesh = plsc.VectorSubcoreMesh(core_axis_name="c", subcore_axis_name="s")

  @functools.partial(
      pl.kernel,
      out_type=(
          jax.ShapeDtypeStruct((NC, n_pad, in_ch), jnp.float32),
          jax.ShapeDtypeStruct((NW * n_pad,), jnp.float32),
      ),
      mesh=mesh,
      compiler_params=pltpu.CompilerParams(needs_layout_passes=False),
      scratch_types=[
          pltpu.VMEM_SHARED((n_pad, in_ch), jnp.float32),
          pltpu.VMEM((2 * IBLK, CHUNK), jnp.int32),
          pltpu.VMEM((CHUNK, in_ch), jnp.float32),
          pltpu.VMEM((CHUNK, in_ch), jnp.float32),
          pltpu.VMEM((n_pad,), jnp.float32),
          pltpu.SemaphoreType.DMA,
          pltpu.SemaphoreType.DMA,
          pltpu.SemaphoreType.DMA,
          pltpu.SemaphoreType.DMA,
      ],
  )
  def agg_kernel(x_hbm, idx_hbm, zeros_hbm, out_hbm, deg_hbm,
                 acc, idx_v, rows_a, rows_b, deg_v, sem_a, sem_b,
                 ssem_a, ssem_b):
    c = lax.axis_index("c")
    s = lax.axis_index("s")
    wid = c * NS + s

    pltpu.sync_copy(zeros_hbm, acc.at[pl.ds(s * rows_per_sub, rows_per_sub)])

    zero16 = jnp.zeros((16,), jnp.float32)

    def zstep(i, carry):
      deg_v[pl.ds(i * 16, 16)] = zero16
      return carry

    lax.fori_loop(0, n_pad // 16, zstep, 0)
    plsc.subcore_barrier()

    one16 = jnp.ones((16,), jnp.float32)

    def scatter_deg(i):
      for j in range(CHUNK // 16):
        idx = idx_v[IBLK + i, pl.ds(j * 16, 16)]
        plsc.addupdate_scatter(deg_v, [idx], one16)
      rem = CHUNK % 16
      if rem:
        idx = idx_v[IBLK + i, pl.ds(CHUNK - 16, 16)]
        mask = lax.iota(jnp.int32, 16) >= (16 - rem)
        plsc.addupdate_scatter(deg_v, [idx], one16, mask=mask)

    bufs = [(rows_a, sem_a, ssem_a), (rows_b, sem_b, ssem_b)]

    def outer(o, carry):
      pltpu.sync_copy(idx_hbm.at[wid, o], idx_v)

      gd = [None] * IBLK
      sd = [None] * IBLK
      rv0, gs0, _ = bufs[0]
      rv1, gs1, _ = bufs[1]
      gd[0] = pltpu.async_copy(x_hbm.at[idx_v.at[0]], rv0, gs0)
      gd[1] = pltpu.async_copy(x_hbm.at[idx_v.at[1]], rv1, gs1)
      for i in range(IBLK):
        rv, gs, ss = bufs[i % 2]
        gd[i].wait()
        sd[i] = pltpu.async_copy(rv, acc.at[idx_v.at[IBLK + i]], ss,
                                 add=True)
        scatter_deg(i)
        if i + 2 < IBLK:
          sd[i].wait()
          gd[i + 2] = pltpu.async_copy(x_hbm.at[idx_v.at[i + 2]], rv, gs)
      sd[IBLK - 2].wait()
      sd[IBLK - 1].wait()
      return carry

    lax.fori_loop(0, n_outer, outer, 0)
    plsc.subcore_barrier()

    row0 = s * rows_per_sub
    pltpu.sync_copy(acc.at[pl.ds(row0, rows_per_sub)],
                    out_hbm.at[c, pl.ds(row0, rows_per_sub)])
    pltpu.sync_copy(deg_v, deg_hbm.at[pl.ds(wid * n_pad, n_pad)])

  return agg_kernel(x, idx_r, zeros)


def _tc_xr_kernel(x_ref, wr_ref, b_ref, o_ref):
  o_ref[...] = jnp.dot(x_ref[...], wr_ref[...],
                       preferred_element_type=jnp.float32) + b_ref[...]


def _tc_combine_kernel(p0_ref, p1_ref, d_ref, xr_ref, wl_ref, o_ref):
  ssum = p0_ref[0] + p1_ref[0]
  deg = jnp.sum(d_ref[...], axis=1, keepdims=True)
  mean = ssum / jnp.maximum(deg, 1.0)
  o_ref[...] = xr_ref[...] + jnp.dot(
      mean, wl_ref[...], preferred_element_type=jnp.float32)


def kernel(x, edge_index, W_l, W_r, b):
  n_nodes, in_ch = x.shape
  n_edges = edge_index.shape[1]
  hid_ch = W_l.shape[0]
  assert in_ch == 128 and hid_ch == 128
  assert n_edges % (NW * CHUNK * IBLK) == 0
  n_iter = n_edges // (NW * CHUNK)
  n_pad = ((n_nodes + 8 * NS - 1) // (8 * NS)) * (8 * NS)
  while n_pad % DEG_W:
    n_pad += 8 * NS

  x = x.astype(jnp.float32)
  ei = edge_index.astype(jnp.int32)
  idx_r = ei.reshape(2, NW, n_iter // IBLK, IBLK, CHUNK).transpose(
      1, 2, 0, 3, 4).reshape(NW, n_iter // IBLK, 2 * IBLK, CHUNK)
  zeros = jnp.zeros((n_pad // NS, in_ch), jnp.float32)

  blk = 1000
  grid = (n_nodes // blk,)
  xr = pl.pallas_call(
      _tc_xr_kernel,
      grid=grid,
      in_specs=[
          pl.BlockSpec((blk, in_ch), lambda i: (i, 0)),
          pl.BlockSpec((in_ch, hid_ch), lambda i: (0, 0)),
          pl.BlockSpec((1, hid_ch), lambda i: (0, 0)),
      ],
      out_specs=pl.BlockSpec((blk, hid_ch), lambda i: (i, 0)),
      out_shape=jax.ShapeDtypeStruct((n_nodes, hid_ch), jnp.float32),
  )(x, W_r.T, b.reshape(1, hid_ch))

  partial, deg = _sc_aggregate(x, idx_r, zeros,
                               n_pad=n_pad, n_iter=n_iter)
  partial = partial.reshape(NC, n_pad, in_ch)[:, :n_nodes]
  deg = deg.reshape(NW, n_pad)[:, :n_nodes].T

  out = pl.pallas_call(
      _tc_combine_kernel,
      grid=grid,
      in_specs=[
          pl.BlockSpec((1, blk, in_ch), lambda i: (0, i, 0)),
          pl.BlockSpec((1, blk, in_ch), lambda i: (1, i, 0)),
          pl.BlockSpec((blk, NW), lambda i: (i, 0)),
          pl.BlockSpec((blk, in_ch), lambda i: (i, 0)),
          pl.BlockSpec((in_ch, hid_ch), lambda i: (0, 0)),
      ],
      out_specs=pl.BlockSpec((blk, hid_ch), lambda i: (i, 0)),
      out_shape=jax.ShapeDtypeStruct((n_nodes, hid_ch), jnp.float32),
  )(partial, partial, deg, xr, W_l.T)
  return out


# speedup vs baseline: 1.0437x; 1.0437x over previous
import functools

import jax
import jax.numpy as jnp
from jax import lax
from jax.experimental import pallas as pl
from jax.experimental.pallas import tpu as pltpu
from jax.experimental.pallas import tpu_sc as plsc

NC = 2
NS = 16
NW = NC * NS

CHUNK = 125
IBLK = 8
DEG_W = 128


def _sc_aggregate(x, idx_r, zeros, *, n_pad, n_iter):
  in_ch = x.shape[1]
  rows_per_sub = n_pad // NS
  n_outer = n_iter // IBLK

  mesh = plsc.VectorSubcoreMesh(core_axis_name="c", subcore_axis_name="s")

  @functools.partial(
      pl.kernel,
      out_type=(
          jax.ShapeDtypeStruct((NC, n_pad, in_ch), jnp.float32),
          jax.ShapeDtypeStruct((NW * n_pad,), jnp.float32),
      ),
      mesh=mesh,
      compiler_params=pltpu.CompilerParams(needs_layout_passes=False),
      scratch_types=[
          pltpu.VMEM_SHARED((n_pad, in_ch), jnp.float32),
          pltpu.VMEM((2 * IBLK, CHUNK), jnp.int32),
          pltpu.VMEM((CHUNK, in_ch), jnp.float32),
          pltpu.VMEM((CHUNK, in_ch), jnp.float32),
          pltpu.VMEM((n_pad,), jnp.float32),
          pltpu.SemaphoreType.DMA,
          pltpu.SemaphoreType.DMA,
          pltpu.SemaphoreType.DMA,
          pltpu.SemaphoreType.DMA,
      ],
  )
  def agg_kernel(x_hbm, idx_hbm, zeros_hbm, out_hbm, deg_hbm,
                 acc, idx_v, rows_a, rows_b, deg_v, sem_a, sem_b,
                 ssem_a, ssem_b):
    c = lax.axis_index("c")
    s = lax.axis_index("s")
    wid = c * NS + s

    pltpu.sync_copy(zeros_hbm, acc.at[pl.ds(s * rows_per_sub, rows_per_sub)])

    zero16 = jnp.zeros((16,), jnp.float32)

    def zstep(i, carry):
      deg_v[pl.ds(i * 16, 16)] = zero16
      return carry

    lax.fori_loop(0, n_pad // 16, zstep, 0)
    plsc.subcore_barrier()

    one16 = jnp.ones((16,), jnp.float32)

    def scatter_deg(i):
      for j in range(CHUNK // 16):
        idx = idx_v[IBLK + i, pl.ds(j * 16, 16)]
        plsc.addupdate_scatter(deg_v, [idx], one16)
      rem = CHUNK % 16
      if rem:
        idx = idx_v[IBLK + i, pl.ds(CHUNK - 16, 16)]
        mask = lax.iota(jnp.int32, 16) >= (16 - rem)
        plsc.addupdate_scatter(deg_v, [idx], one16, mask=mask)

    bufs = [(rows_a, sem_a, ssem_a), (rows_b, sem_b, ssem_b)]

    def outer(o, carry):
      pltpu.sync_copy(idx_hbm.at[wid, o], idx_v)

      gd = [None] * IBLK
      sd = [None] * IBLK
      rv0, gs0, _ = bufs[0]
      rv1, gs1, _ = bufs[1]
      gd[0] = pltpu.async_copy(x_hbm.at[idx_v.at[0]], rv0, gs0)
      gd[1] = pltpu.async_copy(x_hbm.at[idx_v.at[1]], rv1, gs1)
      for i in range(IBLK):
        rv, gs, ss = bufs[i % 2]
        gd[i].wait()
        sd[i] = pltpu.async_copy(rv, acc.at[idx_v.at[IBLK + i]], ss,
                                 add=True)
        scatter_deg(i)
        if i + 2 < IBLK:
          sd[i].wait()
          gd[i + 2] = pltpu.async_copy(x_hbm.at[idx_v.at[i + 2]], rv, gs)
      sd[IBLK - 2].wait()
      sd[IBLK - 1].wait()
      return carry

    lax.fori_loop(0, n_outer, outer, 0)
    plsc.subcore_barrier()

    row0 = s * rows_per_sub
    pltpu.sync_copy(acc.at[pl.ds(row0, rows_per_sub)],
                    out_hbm.at[c, pl.ds(row0, rows_per_sub)])
    pltpu.sync_copy(deg_v, deg_hbm.at[pl.ds(wid * n_pad, n_pad)])

  return agg_kernel(x, idx_r, zeros)


def _tc_combine_kernel(p0_ref, p1_ref, d_ref, x_ref, wl_ref, wr_ref,
                       b_ref, o_ref):
  ssum = p0_ref[0] + p1_ref[0]
  deg = jnp.sum(d_ref[...], axis=1, keepdims=True)
  mean = ssum / jnp.maximum(deg, 1.0)
  acc = jnp.dot(mean, wl_ref[...], preferred_element_type=jnp.float32)
  acc = acc + jnp.dot(x_ref[...], wr_ref[...],
                      preferred_element_type=jnp.float32)
  o_ref[...] = acc + b_ref[...]


def kernel(x, edge_index, W_l, W_r, b):
  n_nodes, in_ch = x.shape
  n_edges = edge_index.shape[1]
  hid_ch = W_l.shape[0]
  assert in_ch == 128 and hid_ch == 128
  assert n_edges % (NW * CHUNK * IBLK) == 0
  n_iter = n_edges // (NW * CHUNK)
  n_pad = ((n_nodes + 8 * NS - 1) // (8 * NS)) * (8 * NS)
  while n_pad % DEG_W:
    n_pad += 8 * NS

  x = x.astype(jnp.float32)
  ei = edge_index.astype(jnp.int32)
  idx_r = ei.reshape(2, NW, n_iter // IBLK, IBLK, CHUNK).transpose(
      1, 2, 0, 3, 4).reshape(NW, n_iter // IBLK, 2 * IBLK, CHUNK)
  zeros = jnp.zeros((n_pad // NS, in_ch), jnp.float32)

  partial, deg = _sc_aggregate(x, idx_r, zeros,
                               n_pad=n_pad, n_iter=n_iter)
  deg = deg.reshape(NW, n_pad)[:, :n_nodes].T

  blk = 1000
  grid = (n_nodes // blk,)
  out = pl.pallas_call(
      _tc_combine_kernel,
      grid=grid,
      in_specs=[
          pl.BlockSpec((1, blk, in_ch), lambda i: (0, i, 0)),
          pl.BlockSpec((1, blk, in_ch), lambda i: (1, i, 0)),
          pl.BlockSpec((blk, NW), lambda i: (i, 0)),
          pl.BlockSpec((blk, in_ch), lambda i: (i, 0)),
          pl.BlockSpec((in_ch, hid_ch), lambda i: (0, 0)),
          pl.BlockSpec((in_ch, hid_ch), lambda i: (0, 0)),
          pl.BlockSpec((1, hid_ch), lambda i: (0, 0)),
      ],
      out_specs=pl.BlockSpec((blk, hid_ch), lambda i: (i, 0)),
      out_shape=jax.ShapeDtypeStruct((n_nodes, hid_ch), jnp.float32),
  )(partial, partial, deg, x, W_l.T, W_r.T, b.reshape(1, hid_ch))
  return out
